# baseline (speedup 1.0000x reference)
import jax
import jax.numpy as jnp
from jax import lax
from jax.experimental import pallas as pl
from jax.experimental.pallas import tpu as pltpu
from jax.experimental.pallas import tpu_sc as plsc

BATCH = 16384
EMBED = 64
NUM_CORES = 2
NUM_SUBCORES = 16
LANES = 16
NUM_WORKERS = NUM_CORES * NUM_SUBCORES
CHUNK = BATCH // NUM_WORKERS
GROUPS = CHUNK // LANES


def _bpr_body(uidx_hbm, pidx_hbm, nidx_hbm, user_t_hbm, item_t_hbm, out_hbm,
              uidx, pidx, nidx, uvals, pvals, nvals, outv, sem):
    wid = lax.axis_index("s") * NUM_CORES + lax.axis_index("c")
    base = wid * CHUNK

    pltpu.sync_copy(uidx_hbm.at[pl.ds(base, CHUNK)], uidx)
    pltpu.sync_copy(pidx_hbm.at[pl.ds(base, CHUNK)], pidx)
    pltpu.sync_copy(nidx_hbm.at[pl.ds(base, CHUNK)], nidx)

    def fire(d):
        pltpu.async_copy(user_t_hbm.at[d].at[uidx], uvals.at[d], sem)
        pltpu.async_copy(item_t_hbm.at[d].at[pidx], pvals.at[d], sem)
        pltpu.async_copy(item_t_hbm.at[d].at[nidx], nvals.at[d], sem)

    def drain(d):
        pltpu.make_async_copy(user_t_hbm.at[d].at[uidx], uvals.at[d], sem).wait()
        pltpu.make_async_copy(item_t_hbm.at[d].at[pidx], pvals.at[d], sem).wait()
        pltpu.make_async_copy(item_t_hbm.at[d].at[nidx], nvals.at[d], sem).wait()

    fire(0)

    @pl.loop(0, EMBED - 1)
    def _pipe(d):
        fire(d + 1)
        drain(d)

    drain(EMBED - 1)

    @pl.loop(0, GROUPS)
    def _dot(g):
        sl = pl.ds(g * LANES, LANES)
        acc = jnp.zeros((LANES,), jnp.float32)
        for d in range(EMBED):
            acc = acc + uvals[d, sl] * (pvals[d, sl] - nvals[d, sl])
        outv[sl] = acc

    pltpu.sync_copy(outv, out_hbm.at[pl.ds(base, CHUNK)])


@jax.jit
def _bpr_sc(uidx, pidx, nidx, user_t, item_t):
    mesh = plsc.VectorSubcoreMesh(core_axis_name="c", subcore_axis_name="s")
    cp = pltpu.CompilerParams(
        needs_layout_passes=False,
        use_tc_tiling_on_sc=False,
    )
    run = pl.kernel(
        _bpr_body,
        out_type=jax.ShapeDtypeStruct((BATCH,), jnp.float32),
        mesh=mesh,
        scratch_types=[
            pltpu.VMEM((CHUNK,), jnp.int32),
            pltpu.VMEM((CHUNK,), jnp.int32),
            pltpu.VMEM((CHUNK,), jnp.int32),
            pltpu.VMEM((EMBED, CHUNK), jnp.float32),
            pltpu.VMEM((EMBED, CHUNK), jnp.float32),
            pltpu.VMEM((EMBED, CHUNK), jnp.float32),
            pltpu.VMEM((CHUNK,), jnp.float32),
            pltpu.SemaphoreType.DMA,
        ],
        compiler_params=cp,
    )
    return run(uidx, pidx, nidx, user_t, item_t)


def kernel(batch, user_memory, item_memory):
    return _bpr_sc(batch[:, 0], batch[:, 1], batch[:, 2],
                   user_memory.T, item_memory.T)


# speedup vs baseline: 1.0017x; 1.0017x over previous
import jax
import jax.numpy as jnp
from jax import lax
from jax.experimental import pallas as pl
from jax.experimental.pallas import tpu as pltpu
from jax.experimental.pallas import tpu_sc as plsc

BATCH = 16384
EMBED = 64
VOCAB = 1000000
NUM_CORES = 2
NUM_SUBCORES = 16
LANES = 16
NUM_WORKERS = NUM_CORES * NUM_SUBCORES
CHUNK = BATCH // NUM_WORKERS
GROUPS = CHUNK // LANES


def _bpr_body(uidx_hbm, pidx_hbm, nidx_hbm, user_f_hbm, item_f_hbm, out_hbm,
              uidx, pidx, nidx, uvals, pvals, nvals, outv, sem):
    wid = lax.axis_index("s") * NUM_CORES + lax.axis_index("c")
    base = wid * CHUNK

    pltpu.sync_copy(uidx_hbm.at[pl.ds(base, CHUNK)], uidx)
    pltpu.sync_copy(pidx_hbm.at[pl.ds(base, CHUNK)], pidx)
    pltpu.sync_copy(nidx_hbm.at[pl.ds(base, CHUNK)], nidx)

    def fire(d):
        usl = user_f_hbm.at[pl.ds(d * VOCAB, VOCAB)]
        isl = item_f_hbm.at[pl.ds(d * VOCAB, VOCAB)]
        pltpu.async_copy(usl.at[uidx], uvals.at[d], sem)
        pltpu.async_copy(isl.at[pidx], pvals.at[d], sem)
        pltpu.async_copy(isl.at[nidx], nvals.at[d], sem)

    def drain(d):
        usl = user_f_hbm.at[pl.ds(d * VOCAB, VOCAB)]
        isl = item_f_hbm.at[pl.ds(d * VOCAB, VOCAB)]
        pltpu.make_async_copy(usl.at[uidx], uvals.at[d], sem).wait()
        pltpu.make_async_copy(isl.at[pidx], pvals.at[d], sem).wait()
        pltpu.make_async_copy(isl.at[nidx], nvals.at[d], sem).wait()

    fire(0)

    @pl.loop(0, EMBED - 1)
    def _pipe(d):
        fire(d + 1)
        drain(d)

    drain(EMBED - 1)

    @pl.loop(0, GROUPS)
    def _dot(g):
        sl = pl.ds(g * LANES, LANES)
        acc = jnp.zeros((LANES,), jnp.float32)
        for d in range(EMBED):
            acc = acc + uvals[d, sl] * (pvals[d, sl] - nvals[d, sl])
        outv[sl] = acc

    pltpu.sync_copy(outv, out_hbm.at[pl.ds(base, CHUNK)])


@jax.jit
def _bpr_sc(uidx, pidx, nidx, user_f, item_f):
    mesh = plsc.VectorSubcoreMesh(core_axis_name="c", subcore_axis_name="s")
    cp = pltpu.CompilerParams(
        needs_layout_passes=False,
        use_tc_tiling_on_sc=False,
    )
    run = pl.kernel(
        _bpr_body,
        out_type=jax.ShapeDtypeStruct((BATCH,), jnp.float32),
        mesh=mesh,
        scratch_types=[
            pltpu.VMEM((CHUNK,), jnp.int32),
            pltpu.VMEM((CHUNK,), jnp.int32),
            pltpu.VMEM((CHUNK,), jnp.int32),
            pltpu.VMEM((EMBED, CHUNK), jnp.float32),
            pltpu.VMEM((EMBED, CHUNK), jnp.float32),
            pltpu.VMEM((EMBED, CHUNK), jnp.float32),
            pltpu.VMEM((CHUNK,), jnp.float32),
            pltpu.SemaphoreType.DMA,
        ],
        compiler_params=cp,
    )
    return run(uidx, pidx, nidx, user_f, item_f)


def kernel(batch, user_memory, item_memory):
    return _bpr_sc(batch[:, 0], batch[:, 1], batch[:, 2],
                   user_memory.T.reshape(-1), item_memory.T.reshape(-1))


# speedup vs baseline: 9.0999x; 9.0845x over previous
import dataclasses
import functools

import jax
import jax.numpy as jnp
from jax import lax
from jax.experimental import pallas as pl
from jax.experimental.pallas import tpu as pltpu
from jax.experimental.pallas import tpu_sc as plsc

BATCH = 16384
EMBED = 64
NUM_CORES = 2
NUM_SUBCORES = 16
LANES = 16
NUM_WORKERS = NUM_CORES * NUM_SUBCORES
CHUNK = BATCH // NUM_WORKERS
GROUPS = CHUNK // LANES


def _bpr_body(batch_hbm, user_hbm, item_hbm, out_hbm,
              slab, uidx, pidx, nidx, urows, prows, nrows, outv, sem):
    wid = lax.axis_index("s") * NUM_CORES + lax.axis_index("c")
    base = wid * CHUNK

    pltpu.sync_copy(batch_hbm.at[pl.ds(base, CHUNK)], slab)

    lanes = lax.iota(jnp.int32, LANES)

    @pl.loop(0, GROUPS)
    def _extract(g):
        rows = g * LANES + lanes
        u = plsc.load_gather(slab, [rows, jnp.zeros((LANES,), jnp.int32)])
        p = plsc.load_gather(slab, [rows, jnp.ones((LANES,), jnp.int32)])
        n = plsc.load_gather(slab, [rows, jnp.full((LANES,), 2, jnp.int32)])
        uidx[pl.ds(g * LANES, LANES)] = u
        pidx[pl.ds(g * LANES, LANES)] = p
        nidx[pl.ds(g * LANES, LANES)] = n

    cp_u = pltpu.async_copy(user_hbm.at[uidx], urows, sem)
    cp_p = pltpu.async_copy(item_hbm.at[pidx], prows, sem)
    cp_n = pltpu.async_copy(item_hbm.at[nidx], nrows, sem)
    cp_u.wait()
    cp_p.wait()
    cp_n.wait()

    @pl.loop(0, GROUPS)
    def _dot(g):
        rows = g * LANES + lanes
        acc = jnp.zeros((LANES,), jnp.float32)
        for d in range(EMBED):
            cols = (lanes + d) & (EMBED - 1)
            pv = plsc.load_gather(prows, [rows, cols])
            nv = plsc.load_gather(nrows, [rows, cols])
            uv = plsc.load_gather(urows, [rows, cols])
            acc = acc + uv * (pv - nv)
        outv[pl.ds(g * LANES, LANES)] = acc

    pltpu.sync_copy(outv, out_hbm.at[pl.ds(base, CHUNK)])


@jax.jit
def _bpr_sc(batch, user_memory, item_memory):
    mesh = plsc.VectorSubcoreMesh(core_axis_name="c", subcore_axis_name="s")
    cp = pltpu.CompilerParams(
        needs_layout_passes=False,
        use_tc_tiling_on_sc=False,
    )
    run = pl.kernel(
        _bpr_body,
        out_type=jax.ShapeDtypeStruct((BATCH,), jnp.float32),
        mesh=mesh,
        scratch_types=[
            pltpu.VMEM((CHUNK, 3), jnp.int32),
            pltpu.VMEM((CHUNK,), jnp.int32),
            pltpu.VMEM((CHUNK,), jnp.int32),
            pltpu.VMEM((CHUNK,), jnp.int32),
            pltpu.VMEM((CHUNK, EMBED), jnp.float32),
            pltpu.VMEM((CHUNK, EMBED), jnp.float32),
            pltpu.VMEM((CHUNK, EMBED), jnp.float32),
            pltpu.VMEM((CHUNK,), jnp.float32),
            pltpu.SemaphoreType.DMA,
        ],
        compiler_params=cp,
    )
    return run(batch, user_memory, item_memory)


def kernel(batch, user_memory, item_memory):
    return _bpr_sc(batch, user_memory, item_memory)


# speedup vs baseline: 21.0391x; 2.3120x over previous
import jax
import jax.numpy as jnp
import numpy as np
from jax import lax
from jax.experimental import pallas as pl
from jax.experimental.pallas import tpu as pltpu
from jax.experimental.pallas import tpu_sc as plsc

BATCH = 16384
EMBED = 64
VOCAB = 1000000
NUM_CORES = 2
NUM_SUBCORES = 16
LANES = 16
NUM_WORKERS = NUM_CORES * NUM_SUBCORES
CHUNK = BATCH // NUM_WORKERS
GROUPS = CHUNK // LANES

BLK_V = 131072
N_VBLK = 8
N_DBLK = EMBED // 8
POOL = N_DBLK * N_VBLK * 8 * BLK_V

HALF = EMBED // 2
HWORDS = HALF * CHUNK
TWORDS = EMBED * CHUNK


def _detile_body(in_ref, o_ref):
    o_ref[...] = in_ref[...].reshape(8 * BLK_V)


@jax.jit
def _detile(t):
    return pl.pallas_call(
        _detile_body,
        grid=(N_DBLK, N_VBLK),
        in_specs=[pl.BlockSpec((8, BLK_V), lambda a, c: (a, c))],
        out_specs=pl.BlockSpec((8 * BLK_V,), lambda a, c: (a * N_VBLK + c,)),
        out_shape=jax.ShapeDtypeStruct((POOL,), jnp.float32),
    )(t)


def _bpr_body(gidxu_hbm, gidxp_hbm, gidxn_hbm, uflat_hbm, iflat_hbm, out_hbm,
              idxu, idxp, idxn, vu, vp, vn, outv, sem):
    wid = lax.axis_index("s") * NUM_CORES + lax.axis_index("c")
    base = wid * CHUNK
    slab = wid * TWORDS

    for h in range(2):
        off = slab + h * HWORDS
        pltpu.sync_copy(gidxu_hbm.at[pl.ds(off, HWORDS)], idxu)
        pltpu.sync_copy(gidxp_hbm.at[pl.ds(off, HWORDS)], idxp)
        pltpu.sync_copy(gidxn_hbm.at[pl.ds(off, HWORDS)], idxn)

        cu = pltpu.async_copy(uflat_hbm.at[idxu], vu, sem)
        cp_ = pltpu.async_copy(iflat_hbm.at[idxp], vp, sem)
        cn = pltpu.async_copy(iflat_hbm.at[idxn], vn, sem)
        cu.wait()
        cp_.wait()
        cn.wait()

        if h == 0:
            @pl.loop(0, GROUPS)
            def _dot_a(g):
                sl = pl.ds(g * LANES, LANES)
                acc = jnp.zeros((LANES,), jnp.float32)
                for dd in range(HALF):
                    vsl = pl.ds(dd * CHUNK + g * LANES, LANES)
                    acc = acc + vu[vsl] * (vp[vsl] - vn[vsl])
                outv[sl] = acc
        else:
            @pl.loop(0, GROUPS)
            def _dot_b(g):
                sl = pl.ds(g * LANES, LANES)
                acc = outv[sl]
                for dd in range(HALF):
                    vsl = pl.ds(dd * CHUNK + g * LANES, LANES)
                    acc = acc + vu[vsl] * (vp[vsl] - vn[vsl])
                outv[sl] = acc

    pltpu.sync_copy(outv, out_hbm.at[pl.ds(base, CHUNK)])


@jax.jit
def _bpr_sc(gidxu, gidxp, gidxn, uflat, iflat):
    mesh = plsc.VectorSubcoreMesh(core_axis_name="c", subcore_axis_name="s")
    cp = pltpu.CompilerParams(
        needs_layout_passes=False,
        use_tc_tiling_on_sc=False,
    )
    run = pl.kernel(
        _bpr_body,
        out_type=jax.ShapeDtypeStruct((BATCH,), jnp.float32),
        mesh=mesh,
        scratch_types=[
            pltpu.VMEM((HWORDS,), jnp.int32),
            pltpu.VMEM((HWORDS,), jnp.int32),
            pltpu.VMEM((HWORDS,), jnp.int32),
            pltpu.VMEM((HWORDS,), jnp.float32),
            pltpu.VMEM((HWORDS,), jnp.float32),
            pltpu.VMEM((HWORDS,), jnp.float32),
            pltpu.VMEM((CHUNK,), jnp.float32),
            pltpu.SemaphoreType.DMA,
        ],
        compiler_params=cp,
    )
    return run(gidxu, gidxp, gidxn, uflat, iflat)


def _pool_indices(v):
    b = (v >> 17) * 1048576 + (v & 131071)
    d = jnp.arange(EMBED, dtype=jnp.int32)
    c = (d >> 3) * 8388608 + (d & 7) * 131072
    arr = b.reshape(NUM_WORKERS, 1, CHUNK) + c.reshape(1, EMBED, 1)
    return arr.reshape(-1)


def kernel(batch, user_memory, item_memory):
    uflat = _detile(user_memory.T)
    iflat = _detile(item_memory.T)
    gidxu = _pool_indices(batch[:, 0])
    gidxp = _pool_indices(batch[:, 1])
    gidxn = _pool_indices(batch[:, 2])
    return _bpr_sc(gidxu, gidxp, gidxn, uflat, iflat)


# speedup vs baseline: 21.5656x; 1.0250x over previous
import jax
import jax.numpy as jnp
import numpy as np
from jax import lax
from jax.experimental import pallas as pl
from jax.experimental.pallas import tpu as pltpu
from jax.experimental.pallas import tpu_sc as plsc

BATCH = 16384
EMBED = 64
VOCAB = 1000000
NUM_CORES = 2
NUM_SUBCORES = 16
LANES = 16
NUM_WORKERS = NUM_CORES * NUM_SUBCORES
CHUNK = BATCH // NUM_WORKERS
GROUPS = CHUNK // LANES

BLK_V = 131072
N_VBLK = 8
N_DBLK = EMBED // 8
POOL = N_DBLK * N_VBLK * 8 * BLK_V

HALF = EMBED // 2
HWORDS = HALF * CHUNK
TWORDS = EMBED * CHUNK


def _detile_body(in_ref, o_ref):
    o_ref[...] = in_ref[...].reshape(8 * BLK_V)


@jax.jit
def _detile(t):
    return pl.pallas_call(
        _detile_body,
        grid=(N_DBLK, N_VBLK),
        in_specs=[pl.BlockSpec((8, BLK_V), lambda a, c: (a, c))],
        out_specs=pl.BlockSpec((8 * BLK_V,), lambda a, c: (a * N_VBLK + c,)),
        out_shape=jax.ShapeDtypeStruct((POOL,), jnp.float32),
    )(t)


def _ugather_body(gidxu_hbm, uflat_hbm, uvals_hbm, idxu, vu, sem):
    wid = lax.axis_index("s") * NUM_CORES + lax.axis_index("c")
    slab = wid * TWORDS

    for h in range(2):
        off = slab + h * HWORDS
        pltpu.sync_copy(gidxu_hbm.at[pl.ds(off, HWORDS)], idxu)
        pltpu.async_copy(uflat_hbm.at[idxu], vu, sem).wait()
        pltpu.sync_copy(vu, uvals_hbm.at[pl.ds(off, HWORDS)])


@jax.jit
def _ugather(gidxu, uflat):
    mesh = plsc.VectorSubcoreMesh(core_axis_name="c", subcore_axis_name="s")
    cp = pltpu.CompilerParams(
        needs_layout_passes=False,
        use_tc_tiling_on_sc=False,
    )
    run = pl.kernel(
        _ugather_body,
        out_type=jax.ShapeDtypeStruct((EMBED * BATCH,), jnp.float32),
        mesh=mesh,
        scratch_types=[
            pltpu.VMEM((HWORDS,), jnp.int32),
            pltpu.VMEM((HWORDS,), jnp.float32),
            pltpu.SemaphoreType.DMA,
        ],
        compiler_params=cp,
    )
    return run(gidxu, uflat)


def _bpr_body(gidxp_hbm, gidxn_hbm, uvals_hbm, iflat_hbm, out_hbm,
              idxp, idxn, vu, vp, vn, outv, sem):
    wid = lax.axis_index("s") * NUM_CORES + lax.axis_index("c")
    base = wid * CHUNK
    slab = wid * TWORDS

    for h in range(2):
        off = slab + h * HWORDS
        pltpu.sync_copy(gidxp_hbm.at[pl.ds(off, HWORDS)], idxp)
        pltpu.sync_copy(gidxn_hbm.at[pl.ds(off, HWORDS)], idxn)

        cu = pltpu.async_copy(uvals_hbm.at[pl.ds(off, HWORDS)], vu, sem)
        cp_ = pltpu.async_copy(iflat_hbm.at[idxp], vp, sem)
        cn = pltpu.async_copy(iflat_hbm.at[idxn], vn, sem)
        cu.wait()
        cp_.wait()
        cn.wait()

        if h == 0:
            @pl.loop(0, GROUPS)
            def _dot_a(g):
                sl = pl.ds(g * LANES, LANES)
                acc = jnp.zeros((LANES,), jnp.float32)
                for dd in range(HALF):
                    vsl = pl.ds(dd * CHUNK + g * LANES, LANES)
                    acc = acc + vu[vsl] * (vp[vsl] - vn[vsl])
                outv[sl] = acc
        else:
            @pl.loop(0, GROUPS)
            def _dot_b(g):
                sl = pl.ds(g * LANES, LANES)
                acc = outv[sl]
                for dd in range(HALF):
                    vsl = pl.ds(dd * CHUNK + g * LANES, LANES)
                    acc = acc + vu[vsl] * (vp[vsl] - vn[vsl])
                outv[sl] = acc

    pltpu.sync_copy(outv, out_hbm.at[pl.ds(base, CHUNK)])


@jax.jit
def _bpr_sc(gidxp, gidxn, uvals, iflat):
    mesh = plsc.VectorSubcoreMesh(core_axis_name="c", subcore_axis_name="s")
    cp = pltpu.CompilerParams(
        needs_layout_passes=False,
        use_tc_tiling_on_sc=False,
    )
    run = pl.kernel(
        _bpr_body,
        out_type=jax.ShapeDtypeStruct((BATCH,), jnp.float32),
        mesh=mesh,
        scratch_types=[
            pltpu.VMEM((HWORDS,), jnp.int32),
            pltpu.VMEM((HWORDS,), jnp.int32),
            pltpu.VMEM((HWORDS,), jnp.float32),
            pltpu.VMEM((HWORDS,), jnp.float32),
            pltpu.VMEM((HWORDS,), jnp.float32),
            pltpu.VMEM((CHUNK,), jnp.float32),
            pltpu.SemaphoreType.DMA,
        ],
        compiler_params=cp,
    )
    return run(gidxp, gidxn, uvals, iflat)


def _pool_indices(v):
    b = (v >> 17) * 1048576 + (v & 131071)
    d = jnp.arange(EMBED, dtype=jnp.int32)
    c = (d >> 3) * 8388608 + (d & 7) * 131072
    arr = b.reshape(NUM_WORKERS, 1, CHUNK) + c.reshape(1, EMBED, 1)
    return arr.reshape(-1)


def kernel(batch, user_memory, item_memory):
    gidxu = _pool_indices(batch[:, 0])
    gidxp = _pool_indices(batch[:, 1])
    gidxn = _pool_indices(batch[:, 2])
    uflat = _detile(user_memory.T)
    uvals = _ugather(gidxu, uflat)
    iflat = _detile(item_memory.T)
    return _bpr_sc(gidxp, gidxn, uvals, iflat)
